# baseline (speedup 1.0000x reference)
import jax
import jax.numpy as jnp
from jax.experimental import pallas as pl
from jax.experimental.pallas import tpu as pltpu

_H = 10
_ITERS = 4
_B = 16
_C = 500
_K = 50
_A_DIM = 12
_D = 128
_CP = 500
_NB = 2
_BB = _B // _NB
_RB = _BB * _CP
_HA = _H * _A_DIM
def _dotd(a, b):
    return jax.lax.dot_general(a, b, (((1,), (0,)), ((), ())),
                               precision=jax.lax.Precision.DEFAULT,
                               preferred_element_type=jnp.float32)


def _doth(a, b):
    return jax.lax.dot_general(a, b, (((1,), (0,)), ((), ())),
                               precision=jax.lax.Precision.HIGHEST,
                               preferred_element_type=jnp.float32)


def _cem_iter_kernel(smp_ref, s0_ref, A_ref, Bm_ref, w_ref, out_ref,
                     mean_ref, std_ref):
    it = pl.program_id(0)
    jj = pl.program_id(1)

    @pl.when(it == 0)
    def _init():
        mean_ref[jj] = jnp.zeros((_BB, _HA), jnp.float32)
        std_ref[jj] = jnp.ones((_BB, _HA), jnp.float32)

    mean = mean_ref[jj]
    std = std_ref[jj]
    mean_bc = jnp.broadcast_to(
        mean[:, None, :], (_BB, _CP, _HA)).reshape(_RB, _HA)
    std_bc = jnp.broadcast_to(
        std[:, None, :], (_BB, _CP, _HA)).reshape(_RB, _HA)
    acts = jnp.clip(mean_bc + std_bc * smp_ref[0, 0], -1.0, 1.0)

    A = A_ref[...]
    Bm = Bm_ref[...]
    s0 = s0_ref[0]
    s = jnp.broadcast_to(s0[:, None, :], (_BB, _CP, _D)).reshape(_RB, _D)
    w = w_ref[...]
    returns = jnp.zeros((_RB, 1), jnp.float32)
    for t in range(_H):
        a_t = acts[:, t * _A_DIM:(t + 1) * _A_DIM]
        s = jnp.tanh(_dotd(s, A) + _dotd(a_t, Bm))
        returns = returns + _dotd(s, w)

    rret = returns.reshape(_BB, _CP, 1)
    r2 = jnp.concatenate(
        [jnp.transpose(rret[b], (1, 0)) for b in range(_BB)], axis=0)

    cidx = jax.lax.broadcasted_iota(jnp.int32, (_BB, _CP), 1)

    kbits = jax.lax.bitcast_convert_type(r2, jnp.int32)
    key = jnp.where(kbits < 0, kbits ^ jnp.int32(0x7FFFFFFF), kbits)
    imin = jnp.int32(-2147483648)
    key = jnp.where(cidx < _C, key, imin)
    ukey = key ^ imin
    p = jnp.zeros((_BB, 1), jnp.int32)
    need = jnp.full((_BB, 1), _K, jnp.int32)
    for bit in range(28, -1, -4):
        hm = jnp.int32(0) if bit == 28 else jnp.int32(-(1 << (bit + 4)))
        matched = (ukey & hm) == p
        nib = jax.lax.shift_right_logical(ukey, bit) & jnp.int32(15)
        cnts = [jnp.sum((matched & (nib == jnp.int32(v))).astype(jnp.int32),
                        axis=1, keepdims=True) for v in range(1, 16)]
        suf = [None] * 17
        suf[16] = jnp.zeros((_BB, 1), jnp.int32)
        for v in range(15, 0, -1):
            suf[v] = suf[v + 1] + cnts[v - 1]
        vstar = jnp.zeros((_BB, 1), jnp.int32)
        for v in range(1, 16):
            vstar = vstar + (suf[v] >= need).astype(jnp.int32)
        sub = jnp.zeros((_BB, 1), jnp.int32)
        for v in range(1, 16):
            sub = sub + jnp.where(jnp.int32(v) > vstar, cnts[v - 1], 0)
        p = p | jax.lax.shift_left(vstar, bit)
        need = need - sub
    kth = p ^ imin
    gt = key > kth
    eq = key == kth
    need_eq = (_K - jnp.sum(gt.astype(jnp.int32), axis=1,
                            keepdims=True)).astype(jnp.float32)
    jidx = jax.lax.broadcasted_iota(jnp.int32, (_CP, _CP), 0)
    iidx = jax.lax.broadcasted_iota(jnp.int32, (_CP, _CP), 1)
    ltri = (jidx <= iidx).astype(jnp.float32)
    tie_rank = _dotd(eq.astype(jnp.float32), ltri)
    maskf = jnp.where(
        gt | (eq & (tie_rank <= need_eq)), 1.0, 0.0)

    ridx = jax.lax.broadcasted_iota(jnp.int32, (_BB, _RB), 1)
    bidx = jax.lax.broadcasted_iota(jnp.int32, (_BB, _RB), 0)
    tiled = jnp.concatenate([maskf] * _BB, axis=1)
    lo = bidx * _CP
    mblk = jnp.where((ridx >= lo) & (ridx < lo + _CP), tiled, 0.0)
    asq = acts * acts
    s1 = _doth(mblk, acts)
    s2 = _doth(mblk, asq)

    inv_k = jnp.float32(1.0 / _K)
    mean_new = s1 * inv_k
    var = s2 * inv_k - mean_new * mean_new
    std_new = jnp.sqrt(jnp.maximum(var, 0.0))

    mean_ref[jj] = mean_new
    std_ref[jj] = std_new
    out_ref[0] = mean_new[:, :_A_DIM]


def kernel(s0, A, Bm, w):
    noise_key = jax.random.key(42)
    smp = jnp.stack([
        jax.random.normal(jax.random.fold_in(noise_key, i),
                          (_H, _B, _C, _A_DIM), dtype=jnp.float32)
        for i in range(_ITERS)])
    smp = smp.reshape(_ITERS, _H, _B * _C, _A_DIM)
    smp = jnp.transpose(smp, (0, 2, 1, 3))
    smp = smp.reshape(_ITERS, _NB, _RB, _HA)

    out = pl.pallas_call(
        _cem_iter_kernel,
        grid=(_ITERS, _NB),
        in_specs=[
            pl.BlockSpec((1, 1, _RB, _HA), lambda i, j: (i, j, 0, 0)),
            pl.BlockSpec((1, _BB, _D), lambda i, j: (j, 0, 0)),
            pl.BlockSpec((_D, _D), lambda i, j: (0, 0)),
            pl.BlockSpec((_A_DIM, _D), lambda i, j: (0, 0)),
            pl.BlockSpec((_D, 1), lambda i, j: (0, 0)),
        ],
        out_specs=pl.BlockSpec((1, _BB, _A_DIM), lambda i, j: (j, 0, 0)),
        out_shape=jax.ShapeDtypeStruct((_NB, _BB, _A_DIM), jnp.float32),
        scratch_shapes=[
            pltpu.VMEM((_NB, _BB, _HA), jnp.float32),
            pltpu.VMEM((_NB, _BB, _HA), jnp.float32),
        ],
        compiler_params=pltpu.CompilerParams(
            dimension_semantics=("arbitrary", "arbitrary")),
    )(smp, s0.reshape(_NB, _BB, _D), A, Bm, w.reshape(_D, 1))
    return out.reshape(_B, _A_DIM)


# speedup vs baseline: 1.3844x; 1.3844x over previous
import jax
import jax.numpy as jnp
from jax.experimental import pallas as pl
from jax.experimental.pallas import tpu as pltpu

_H = 10
_ITERS = 4
_B = 16
_C = 500
_K = 50
_A_DIM = 12
_D = 128
_CP = 512
_NB = 2
_BB = _B // _NB
_RB = _BB * _CP
_HA = _H * _A_DIM
def _dotd(a, b):
    return jax.lax.dot_general(a, b, (((1,), (0,)), ((), ())),
                               precision=jax.lax.Precision.DEFAULT,
                               preferred_element_type=jnp.float32)


def _doth(a, b):
    return jax.lax.dot_general(a, b, (((1,), (0,)), ((), ())),
                               precision=jax.lax.Precision.HIGHEST,
                               preferred_element_type=jnp.float32)


def _cem_iter_kernel(smp_ref, s0_ref, A_ref, Bm_ref, w_ref, out_ref,
                     mean_ref, std_ref):
    it = pl.program_id(0)
    jj = pl.program_id(1)

    @pl.when(it == 0)
    def _init():
        mean_ref[jj] = jnp.zeros((_BB, _HA), jnp.float32)
        std_ref[jj] = jnp.ones((_BB, _HA), jnp.float32)

    mean = mean_ref[jj]
    std = std_ref[jj]
    mean_bc = jnp.broadcast_to(
        mean[:, None, :], (_BB, _CP, _HA)).reshape(_RB, _HA)
    std_bc = jnp.broadcast_to(
        std[:, None, :], (_BB, _CP, _HA)).reshape(_RB, _HA)
    acts = jnp.clip(mean_bc + std_bc * smp_ref[0, 0], -1.0, 1.0)

    A = A_ref[...]
    Bm = Bm_ref[...]
    s0 = s0_ref[0]
    s = jnp.broadcast_to(s0[:, None, :], (_BB, _CP, _D)).reshape(_RB, _D)
    w = w_ref[...]
    returns = jnp.zeros((_RB, 1), jnp.float32)
    for t in range(_H):
        a_t = acts[:, t * _A_DIM:(t + 1) * _A_DIM]
        s = jnp.tanh(_dotd(s, A) + _dotd(a_t, Bm))
        returns = returns + _dotd(s, w)

    rret = returns.reshape(_BB, _CP, 1)
    r2 = jnp.concatenate(
        [jnp.transpose(rret[b], (1, 0)) for b in range(_BB)], axis=0)

    cidx = jax.lax.broadcasted_iota(jnp.int32, (_BB, _CP), 1)

    kbits = jax.lax.bitcast_convert_type(r2, jnp.int32)
    key = jnp.where(kbits < 0, kbits ^ jnp.int32(0x7FFFFFFF), kbits)
    imin = jnp.int32(-2147483648)
    key = jnp.where(cidx < _C, key, imin)
    ukey = key ^ imin
    p = jnp.zeros((_BB, 1), jnp.int32)
    need = jnp.full((_BB, 1), _K, jnp.int32)
    for bit in range(28, -1, -4):
        hm = jnp.int32(0) if bit == 28 else jnp.int32(-(1 << (bit + 4)))
        matched = (ukey & hm) == p
        nib = jax.lax.shift_right_logical(ukey, bit) & jnp.int32(15)
        cnts = [jnp.sum((matched & (nib == jnp.int32(v))).astype(jnp.int32),
                        axis=1, keepdims=True) for v in range(1, 16)]
        suf = [None] * 17
        suf[16] = jnp.zeros((_BB, 1), jnp.int32)
        for v in range(15, 0, -1):
            suf[v] = suf[v + 1] + cnts[v - 1]
        vstar = jnp.zeros((_BB, 1), jnp.int32)
        for v in range(1, 16):
            vstar = vstar + (suf[v] >= need).astype(jnp.int32)
        sub = jnp.zeros((_BB, 1), jnp.int32)
        for v in range(1, 16):
            sub = sub + jnp.where(jnp.int32(v) > vstar, cnts[v - 1], 0)
        p = p | jax.lax.shift_left(vstar, bit)
        need = need - sub
    kth = p ^ imin
    gt = key > kth
    eq = key == kth
    need_eq = (_K - jnp.sum(gt.astype(jnp.int32), axis=1,
                            keepdims=True)).astype(jnp.float32)
    jidx = jax.lax.broadcasted_iota(jnp.int32, (_CP, _CP), 0)
    iidx = jax.lax.broadcasted_iota(jnp.int32, (_CP, _CP), 1)
    ltri = (jidx <= iidx).astype(jnp.float32)
    tie_rank = _dotd(eq.astype(jnp.float32), ltri)
    maskf = jnp.where(
        gt | (eq & (tie_rank <= need_eq)), 1.0, 0.0)

    ridx = jax.lax.broadcasted_iota(jnp.int32, (_BB, _RB), 1)
    bidx = jax.lax.broadcasted_iota(jnp.int32, (_BB, _RB), 0)
    tiled = jnp.concatenate([maskf] * _BB, axis=1)
    mblk = jnp.where((ridx >> 9) == bidx, tiled, 0.0)
    asq = acts * acts
    s1 = _doth(mblk, acts)
    s2 = _doth(mblk, asq)

    inv_k = jnp.float32(1.0 / _K)
    mean_new = s1 * inv_k
    var = s2 * inv_k - mean_new * mean_new
    std_new = jnp.sqrt(jnp.maximum(var, 0.0))

    mean_ref[jj] = mean_new
    std_ref[jj] = std_new
    out_ref[0] = mean_new[:, :_A_DIM]


def kernel(s0, A, Bm, w):
    noise_key = jax.random.key(42)
    smp = jnp.stack([
        jax.random.normal(jax.random.fold_in(noise_key, i),
                          (_H, _B, _C, _A_DIM), dtype=jnp.float32)
        for i in range(_ITERS)])
    smp = jnp.transpose(smp, (0, 2, 3, 1, 4))
    smp = jnp.pad(smp, ((0, 0), (0, 0), (0, _CP - _C), (0, 0), (0, 0)))
    smp = smp.reshape(_ITERS, _NB, _RB, _HA)

    out = pl.pallas_call(
        _cem_iter_kernel,
        grid=(_ITERS, _NB),
        in_specs=[
            pl.BlockSpec((1, 1, _RB, _HA), lambda i, j: (i, j, 0, 0)),
            pl.BlockSpec((1, _BB, _D), lambda i, j: (j, 0, 0)),
            pl.BlockSpec((_D, _D), lambda i, j: (0, 0)),
            pl.BlockSpec((_A_DIM, _D), lambda i, j: (0, 0)),
            pl.BlockSpec((_D, 1), lambda i, j: (0, 0)),
        ],
        out_specs=pl.BlockSpec((1, _BB, _A_DIM), lambda i, j: (j, 0, 0)),
        out_shape=jax.ShapeDtypeStruct((_NB, _BB, _A_DIM), jnp.float32),
        scratch_shapes=[
            pltpu.VMEM((_NB, _BB, _HA), jnp.float32),
            pltpu.VMEM((_NB, _BB, _HA), jnp.float32),
        ],
        compiler_params=pltpu.CompilerParams(
            dimension_semantics=("arbitrary", "arbitrary")),
    )(smp, s0.reshape(_NB, _BB, _D), A, Bm, w.reshape(_D, 1))
    return out.reshape(_B, _A_DIM)
